# baseline (speedup 1.0000x reference)
import math

import jax
import jax.numpy as jnp
import numpy as np
from jax.experimental import pallas as pl

B, T, C = 8, 512, 512
NH = 16
HS = C // NH
LEVEL = 3
LMIN = 2 ** LEVEL
NSETS = 127
F2 = 2 * HS
GG = 1


def _build_constants():
    t = np.arange(T)
    ltri = ((t[:, None] // LMIN == t[None, :] // LMIN)
            & (t[None, :] <= t[:, None])).astype(np.float32)
    rows = []
    ends = []
    levelmax = int(math.log2(T))
    for lvl in range(LEVEL, levelmax + 1):
        curlen = 2 ** lvl
        nb = T // curlen
        for j in range(nb):
            r = np.zeros((T,), np.float32)
            r[j * curlen:(j + 1) * curlen] = 1.0
            rows.append(r)
            ends.append((j + 1) * curlen)
    agg = np.stack(rows, axis=0)
    spread = np.kron(np.eye(GG * HS, dtype=np.float32),
                     np.ones((1, HS), np.float32))
    tilem = np.kron(np.eye(GG, dtype=np.float32),
                    np.kron(np.ones((1, HS), np.float32),
                            np.eye(HS, dtype=np.float32)))
    cm = np.concatenate([ltri, agg, np.zeros((1, T), np.float32)], axis=0)
    ends_arr = np.asarray(ends, np.int32)
    m_prefix = (t // LMIN) * LMIN
    maskadd = np.where(ends_arr[None, :] <= m_prefix[:, None],
                       0.0, -1e30).astype(np.float32)

    return (jnp.asarray(cm), jnp.asarray(maskadd), jnp.asarray(spread),
            jnp.asarray(tilem))


def _fused_kernel(x_ref, wqkv_ref, wkvm_ref, bb_ref,
                  wc_ref, cm_ref, maskadd_ref, spread_ref, tilem_ref, out_ref):
    x = x_ref[0]
    f32 = jnp.float32
    scale = f32(1.0 / math.sqrt(HS))

    def elu1(z):
        return jnp.where(z > 0, z + f32(1.0), jnp.exp(jnp.minimum(z, f32(0.0))))

    qkv = jnp.dot(x, wqkv_ref[...], preferred_element_type=f32)
    q = elu1(qkv[:, :C]) * scale
    k = elu1(qkv[:, C:2 * C])
    v = qkv[:, 2 * C:]

    bb = bb_ref[...]
    maskadd = maskadd_ref[...]
    out_parts = []
    for grp in range(NH // GG):
        k4 = k[:, grp * GG * HS:(grp + 1) * GG * HS]
        kr4 = jnp.dot(k4, spread_ref[...], preferred_element_type=f32)
        v4 = v[:, grp * GG * HS:(grp + 1) * GG * HS]
        vt4 = jnp.dot(v4, tilem_ref[...], preferred_element_type=f32)
        outer4 = kr4 * vt4
        g4 = jnp.concatenate(
            [jnp.dot(outer4[:, j * HS * HS:(j + 1) * HS * HS], wkvm_ref[...],
                     preferred_element_type=f32) for j in range(GG)],
            axis=1)
        cg4 = jnp.dot(cm_ref[...], g4, preferred_element_type=f32)
        bb4 = bb[:, grp * GG * F2:(grp + 1) * GG * F2]
        gcum4 = cg4[:T] + bb4
        sets4 = cg4[T:T + NSETS] + bb4
        for j in range(GG):
            h = grp * GG + j
            qh = q[:, h * HS:(h + 1) * HS]
            kset = sets4[:, j * F2:j * F2 + HS]
            vset = sets4[:, j * F2 + HS:(j + 1) * F2]
            ktail = gcum4[:, j * F2:j * F2 + HS]
            vtail = gcum4[:, j * F2 + HS:(j + 1) * F2]
            lg = jax.lax.dot_general(qh, kset, (((1,), (1,)), ((), ())),
                                     preferred_element_type=f32)
            tl = jnp.sum(qh * ktail, axis=1, keepdims=True)
            lg = lg + maskadd
            m = jnp.maximum(jnp.max(lg, axis=1, keepdims=True), tl)
            e = jnp.exp(lg - m)
            etl = jnp.exp(tl - m)
            s = jnp.sum(e, axis=1, keepdims=True) + etl
            oh = (jnp.dot(e, vset, preferred_element_type=f32)
                  + etl * vtail) / s
            out_parts.append(oh)
    out_all = jnp.concatenate(out_parts, axis=1)
    out_ref[0] = jnp.dot(out_all, wc_ref[...], preferred_element_type=f32)


def kernel(x, Wq, Wk, Wv, Wkm, bkm, Wvm, bvm, Wc):
    cm, maskadd, spread, tilem = _build_constants()
    wkvm = jnp.concatenate([Wkm, Wvm], axis=1)
    wqkv = jnp.concatenate([Wq, Wk, Wv], axis=1)
    bb = jnp.tile(jnp.concatenate([bkm, bvm]), (NH,))[None, :]

    full = lambda shp: pl.BlockSpec(shp, lambda b: (0,) * len(shp))
    return pl.pallas_call(
        _fused_kernel,
        grid=(B,),
        in_specs=[
            pl.BlockSpec((1, T, C), lambda b: (b, 0, 0)),
            full((C, 3 * C)),
            full((HS * HS, F2)), full((1, NH * F2)),
            full((C, C)), full((T + NSETS + 1, T)), full((T, NSETS)),
            full((GG * HS, GG * HS * HS)), full((GG * HS, GG * HS * HS)),
        ],
        out_specs=pl.BlockSpec((1, T, C), lambda b: (b, 0, 0)),
        out_shape=jax.ShapeDtypeStruct((B, T, C), jnp.float32),
    )(x, wqkv, wkvm, bb, Wc, cm, maskadd, spread, tilem)


# speedup vs baseline: 1.0184x; 1.0184x over previous
import math

import jax
import jax.numpy as jnp
import numpy as np
from jax.experimental import pallas as pl

B, T, C = 8, 512, 512
NH = 16
HS = C // NH
LEVEL = 3
LMIN = 2 ** LEVEL
NSETS = 127
F2 = 2 * HS
GG = 2


def _build_constants():
    t = np.arange(T)
    ltri = ((t[:, None] // LMIN == t[None, :] // LMIN)
            & (t[None, :] <= t[:, None])).astype(np.float32)
    rows = []
    ends = []
    levelmax = int(math.log2(T))
    for lvl in range(LEVEL, levelmax + 1):
        curlen = 2 ** lvl
        nb = T // curlen
        for j in range(nb):
            r = np.zeros((T,), np.float32)
            r[j * curlen:(j + 1) * curlen] = 1.0
            rows.append(r)
            ends.append((j + 1) * curlen)
    agg = np.stack(rows, axis=0)
    spread = np.kron(np.eye(GG * HS, dtype=np.float32),
                     np.ones((1, HS), np.float32))
    tilem = np.kron(np.eye(GG, dtype=np.float32),
                    np.kron(np.ones((1, HS), np.float32),
                            np.eye(HS, dtype=np.float32)))
    cm = np.concatenate([ltri, agg, np.zeros((1, T), np.float32)], axis=0)
    ends_arr = np.asarray(ends, np.int32)
    m_prefix = (t // LMIN) * LMIN
    maskadd = np.where(ends_arr[None, :] <= m_prefix[:, None],
                       0.0, -1e30).astype(np.float32)

    return (jnp.asarray(cm), jnp.asarray(maskadd), jnp.asarray(spread),
            jnp.asarray(tilem))


def _fused_kernel(x_ref, wqkv_ref, wkvm_ref, bb_ref,
                  wc_ref, cm_ref, maskadd_ref, spread_ref, tilem_ref, out_ref):
    x = x_ref[0]
    f32 = jnp.float32
    scale = f32(1.0 / math.sqrt(HS))

    def elu1(z):
        return jnp.where(z > 0, z + f32(1.0), jnp.exp(jnp.minimum(z, f32(0.0))))

    qkv = jnp.dot(x, wqkv_ref[...], preferred_element_type=f32)
    q = elu1(qkv[:, :C]) * scale
    k = elu1(qkv[:, C:2 * C])
    v = qkv[:, 2 * C:]

    bb = bb_ref[...]
    maskadd = maskadd_ref[...]
    out_parts = []
    for grp in range(NH // GG):
        k4 = k[:, grp * GG * HS:(grp + 1) * GG * HS]
        kr4 = jnp.dot(k4, spread_ref[...], preferred_element_type=f32)
        v4 = v[:, grp * GG * HS:(grp + 1) * GG * HS]
        vt4 = jnp.dot(v4, tilem_ref[...], preferred_element_type=f32)
        outer4 = kr4 * vt4
        g4 = jnp.concatenate(
            [jnp.dot(outer4[:, j * HS * HS:(j + 1) * HS * HS], wkvm_ref[...],
                     preferred_element_type=f32) for j in range(GG)],
            axis=1)
        cg4 = jnp.dot(cm_ref[...], g4, preferred_element_type=f32)
        bb4 = bb[:, grp * GG * F2:(grp + 1) * GG * F2]
        gcum4 = cg4[:T] + bb4
        sets4 = cg4[T:T + NSETS] + bb4
        for j in range(GG):
            h = grp * GG + j
            qh = q[:, h * HS:(h + 1) * HS]
            kset = sets4[:, j * F2:j * F2 + HS]
            vset = sets4[:, j * F2 + HS:(j + 1) * F2]
            ktail = gcum4[:, j * F2:j * F2 + HS]
            vtail = gcum4[:, j * F2 + HS:(j + 1) * F2]
            lg = jax.lax.dot_general(qh, kset, (((1,), (1,)), ((), ())),
                                     preferred_element_type=f32)
            tl = jnp.sum(qh * ktail, axis=1, keepdims=True)
            lg = lg + maskadd
            m = jnp.maximum(jnp.max(lg, axis=1, keepdims=True), tl)
            e = jnp.exp(lg - m)
            etl = jnp.exp(tl - m)
            s = jnp.sum(e, axis=1, keepdims=True) + etl
            oh = (jnp.dot(e, vset, preferred_element_type=f32)
                  + etl * vtail) / s
            out_parts.append(oh)
    out_all = jnp.concatenate(out_parts, axis=1)
    out_ref[0] = jnp.dot(out_all, wc_ref[...], preferred_element_type=f32)


def kernel(x, Wq, Wk, Wv, Wkm, bkm, Wvm, bvm, Wc):
    cm, maskadd, spread, tilem = _build_constants()
    wkvm = jnp.concatenate([Wkm, Wvm], axis=1)
    wqkv = jnp.concatenate([Wq, Wk, Wv], axis=1)
    bb = jnp.tile(jnp.concatenate([bkm, bvm]), (NH,))[None, :]

    full = lambda shp: pl.BlockSpec(shp, lambda b: (0,) * len(shp))
    return pl.pallas_call(
        _fused_kernel,
        grid=(B,),
        in_specs=[
            pl.BlockSpec((1, T, C), lambda b: (b, 0, 0)),
            full((C, 3 * C)),
            full((HS * HS, F2)), full((1, NH * F2)),
            full((C, C)), full((T + NSETS + 1, T)), full((T, NSETS)),
            full((GG * HS, GG * HS * HS)), full((GG * HS, GG * HS * HS)),
        ],
        out_specs=pl.BlockSpec((1, T, C), lambda b: (b, 0, 0)),
        out_shape=jax.ShapeDtypeStruct((B, T, C), jnp.float32),
    )(x, wqkv, wkvm, bb, Wc, cm, maskadd, spread, tilem)
